# baseline (speedup 1.0000x reference)
import functools

import jax
import jax.numpy as jnp
from jax import lax
from jax.experimental import pallas as pl
from jax.experimental.pallas import tpu as pltpu
from jax.experimental.pallas import tpu_sc as plsc

VOCAB = 1000000
SEQ_LEN = 200
EMBED_DIM = 32
BATCH = 4096

NUM_CORES = 2
NUM_SUBCORES = 16
NUM_WORKERS = NUM_CORES * NUM_SUBCORES

LANES = 16
BB = 128
NBLK = BATCH // BB
DB = 8
NG = EMBED_DIM // DB
GRP = 4
NGRP = NBLK // GRP
NPAIR = NGRP // 2

L_BASE = SEQ_LEN // NUM_WORKERS
L_EXTRA = SEQ_LEN % NUM_WORKERS

OUT_ROWS = SEQ_LEN * NG * NBLK


def _body(idx_hbm, tok_hbm, pos_hbm, out_hbm,
          idx_v, r0_v, r1_v, o0_v, o1_v, pos_v, psplat_v,
          gsem0, gsem1, osem0, osem1):
    wid = lax.axis_index("s") * NUM_CORES + lax.axis_index("c")
    pltpu.sync_copy(pos_hbm, pos_v)

    lo = wid * L_BASE + jnp.minimum(wid, L_EXTRA)
    cnt = L_BASE + jnp.where(wid < L_EXTRA, 1, 0)

    iota = lax.iota(jnp.int32, LANES)
    zeros16 = jnp.zeros((LANES,), jnp.int32)
    row_vecs = [iota + (hh * LANES) for hh in range(BB // LANES)]
    d_vecs = [jnp.full((LANES,), d, jnp.int32) for d in range(EMBED_DIM)]

    r_bufs = (r0_v, r1_v)
    o_bufs = (o0_v, o1_v)
    gsems = (gsem0, gsem1)
    osems = (osem0, osem1)

    def fire_group(g, b):
        for j in range(GRP):
            pltpu.async_copy(
                tok_hbm.at[idx_v.at[g * GRP + j]], r_bufs[b].at[j], gsems[b]
            )

    def drain_gathers(b):
        for j in range(GRP):
            pltpu.make_async_copy(
                tok_hbm.at[pl.ds(0, BB)], r_bufs[b].at[j], gsems[b]
            ).wait()

    def drain_outs(b):
        for g_ in range(NG):
            pltpu.make_async_copy(
                out_hbm.at[pl.ds(0, GRP)], o_bufs[b].at[g_], osems[b]
            ).wait()

    def l_body(l, _):
        pltpu.sync_copy(idx_hbm.at[l], idx_v)
        l_vec = zeros16 + l

        for d in range(EMBED_DIM):
            psplat_v[d, :] = plsc.load_gather(pos_v, [l_vec, d_vecs[d]])

        fire_group(0, 0)
        fire_group(1, 1)

        def pair_body(p, carry):
            for b in range(2):
                g = 2 * p + b
                rbuf, obuf = r_bufs[b], o_bufs[b]
                drain_gathers(b)

                @pl.when(p >= 1)
                def _():
                    drain_outs(b)

                def cc_body(cc, carry2):
                    rblk = rbuf.at[cc]
                    for d in range(EMBED_DIM):
                        g_, dd = d // DB, d % DB
                        pv = psplat_v[d, :]
                        for hh in range(BB // LANES):
                            vals = plsc.load_gather(
                                rblk, [row_vecs[hh], d_vecs[d]]
                            )
                            obuf[g_, cc, dd, pl.ds(hh * LANES, LANES)] = (
                                vals + pv
                            )
                    return carry2

                lax.fori_loop(0, GRP, cc_body, 0)

                for g_ in range(NG):
                    row0 = (l * NG + g_) * NBLK + g * GRP
                    pltpu.async_copy(
                        obuf.at[g_], out_hbm.at[pl.ds(row0, GRP)], osems[b]
                    )

                @pl.when(p < NPAIR - 1)
                def _():
                    fire_group(g + 2, b)

            return carry

        lax.fori_loop(0, NPAIR, pair_body, 0)
        drain_outs(0)
        drain_outs(1)
        return _

    lax.fori_loop(lo, lo + cnt, l_body, 0)


_mesh = plsc.VectorSubcoreMesh(core_axis_name="c", subcore_axis_name="s")

_sc_call = functools.partial(
    pl.kernel,
    out_type=jax.ShapeDtypeStruct((OUT_ROWS, DB, BB), jnp.float32),
    mesh=_mesh,
    scratch_types=[
        pltpu.VMEM((NBLK, BB), jnp.int32),
        pltpu.VMEM((GRP, BB, EMBED_DIM), jnp.float32),
        pltpu.VMEM((GRP, BB, EMBED_DIM), jnp.float32),
        pltpu.VMEM((NG, GRP, DB, BB), jnp.float32),
        pltpu.VMEM((NG, GRP, DB, BB), jnp.float32),
        pltpu.VMEM((SEQ_LEN, EMBED_DIM), jnp.float32),
        pltpu.VMEM((EMBED_DIM, LANES), jnp.float32),
        pltpu.SemaphoreType.DMA,
        pltpu.SemaphoreType.DMA,
        pltpu.SemaphoreType.DMA,
        pltpu.SemaphoreType.DMA,
    ],
    compiler_params=pltpu.CompilerParams(
        use_tc_tiling_on_sc=False, needs_layout_passes=False
    ),
)


@jax.jit
def kernel(inputs, token_table, pos_table):
    idx = inputs.astype(jnp.int32).T.reshape(SEQ_LEN, NBLK, BB)
    o3 = _sc_call(_body)(idx, token_table, pos_table)
    o5 = o3.reshape(SEQ_LEN, NG, NBLK, DB, BB)
    out = o5.transpose(0, 1, 3, 2, 4).reshape(SEQ_LEN, EMBED_DIM, BATCH)
    return out.transpose(2, 0, 1)


# speedup vs baseline: 1.5598x; 1.5598x over previous
import functools

import jax
import jax.numpy as jnp
from jax import lax
from jax.experimental import pallas as pl
from jax.experimental.pallas import tpu as pltpu
from jax.experimental.pallas import tpu_sc as plsc

VOCAB = 1000000
SEQ_LEN = 200
EMBED_DIM = 32
BATCH = 4096

NUM_CORES = 2
NUM_SUBCORES = 16
NUM_WORKERS = NUM_CORES * NUM_SUBCORES

LANES = 16
BB = 128
NBLK = BATCH // BB
DB = 8
NG = EMBED_DIM // DB
GRP = 4
NGRP = NBLK // GRP
NPAIR = NGRP // 2

L_BASE = SEQ_LEN // NUM_WORKERS
L_EXTRA = SEQ_LEN % NUM_WORKERS

OUT_ROWS = SEQ_LEN * NG * NBLK


def _body(idx_hbm, tok_hbm, pos_hbm, out_hbm,
          idx_v, r0_v, r1_v, o0_v, o1_v, pos_v, psplat_v,
          gsem0, gsem1, osem0, osem1):
    wid = lax.axis_index("s") * NUM_CORES + lax.axis_index("c")
    pltpu.sync_copy(pos_hbm, pos_v)

    lo = wid * L_BASE + jnp.minimum(wid, L_EXTRA)
    cnt = L_BASE + jnp.where(wid < L_EXTRA, 1, 0)

    iota = lax.iota(jnp.int32, LANES)
    zeros16 = jnp.zeros((LANES,), jnp.int32)
    row_vecs = [iota + (hh * LANES) for hh in range(BB // LANES)]
    d_vecs = [jnp.full((LANES,), d, jnp.int32) for d in range(EMBED_DIM)]

    r_bufs = (r0_v, r1_v)
    o_bufs = (o0_v, o1_v)
    gsems = (gsem0, gsem1)
    osems = (osem0, osem1)

    def fire_group(g, b):
        for j in range(GRP):
            pltpu.async_copy(
                tok_hbm.at[idx_v.at[g * GRP + j]], r_bufs[b].at[j], gsems[b]
            )

    def drain_gathers(b):
        for j in range(GRP):
            pltpu.make_async_copy(
                tok_hbm.at[pl.ds(0, BB)], r_bufs[b].at[j], gsems[b]
            ).wait()

    def drain_outs(b):
        for g_ in range(NG):
            pltpu.make_async_copy(
                out_hbm.at[pl.ds(0, GRP)], o_bufs[b].at[g_], osems[b]
            ).wait()

    def l_body(l, _):
        pltpu.sync_copy(idx_hbm.at[l], idx_v)
        l_vec = zeros16 + l

        for d in range(EMBED_DIM):
            psplat_v[d, :] = plsc.load_gather(pos_v, [l_vec, d_vecs[d]])

        fire_group(0, 0)
        fire_group(1, 1)

        def pair_body(p, carry):
            for b in range(2):
                g = 2 * p + b
                rbuf, obuf = r_bufs[b], o_bufs[b]
                drain_gathers(b)

                @pl.when(p >= 1)
                def _():
                    drain_outs(b)

                @plsc.parallel_loop(0, GRP * EMBED_DIM, unroll=2)
                def _(t):
                    cc = lax.shift_right_logical(t, 5)
                    d = lax.bitwise_and(t, EMBED_DIM - 1)
                    g_ = lax.shift_right_logical(d, 3)
                    dd = lax.bitwise_and(d, DB - 1)
                    rblk = rbuf.at[cc]
                    pv = psplat_v[d, :]
                    d_vec = zeros16 + d
                    for hh in range(BB // LANES):
                        vals = plsc.load_gather(rblk, [row_vecs[hh], d_vec])
                        obuf[g_, cc, dd, pl.ds(hh * LANES, LANES)] = vals + pv

                for g_ in range(NG):
                    row0 = (l * NG + g_) * NBLK + g * GRP
                    pltpu.async_copy(
                        obuf.at[g_], out_hbm.at[pl.ds(row0, GRP)], osems[b]
                    )

                @pl.when(p < NPAIR - 1)
                def _():
                    fire_group(g + 2, b)

            return carry

        lax.fori_loop(0, NPAIR, pair_body, 0)
        drain_outs(0)
        drain_outs(1)
        return _

    lax.fori_loop(lo, lo + cnt, l_body, 0)


_mesh = plsc.VectorSubcoreMesh(core_axis_name="c", subcore_axis_name="s")

_sc_call = functools.partial(
    pl.kernel,
    out_type=jax.ShapeDtypeStruct((OUT_ROWS, DB, BB), jnp.float32),
    mesh=_mesh,
    scratch_types=[
        pltpu.VMEM((NBLK, BB), jnp.int32),
        pltpu.VMEM((GRP, BB, EMBED_DIM), jnp.float32),
        pltpu.VMEM((GRP, BB, EMBED_DIM), jnp.float32),
        pltpu.VMEM((NG, GRP, DB, BB), jnp.float32),
        pltpu.VMEM((NG, GRP, DB, BB), jnp.float32),
        pltpu.VMEM((SEQ_LEN, EMBED_DIM), jnp.float32),
        pltpu.VMEM((EMBED_DIM, LANES), jnp.float32),
        pltpu.SemaphoreType.DMA,
        pltpu.SemaphoreType.DMA,
        pltpu.SemaphoreType.DMA,
        pltpu.SemaphoreType.DMA,
    ],
    compiler_params=pltpu.CompilerParams(
        use_tc_tiling_on_sc=False, needs_layout_passes=False
    ),
)


@jax.jit
def kernel(inputs, token_table, pos_table):
    idx = inputs.astype(jnp.int32).T.reshape(SEQ_LEN, NBLK, BB)
    o3 = _sc_call(_body)(idx, token_table, pos_table)
    o5 = o3.reshape(SEQ_LEN, NG, NBLK, DB, BB)
    out = o5.transpose(0, 1, 3, 2, 4).reshape(SEQ_LEN, EMBED_DIM, BATCH)
    return out.transpose(2, 0, 1)


# speedup vs baseline: 1.5609x; 1.0007x over previous
import functools

import jax
import jax.numpy as jnp
from jax import lax
from jax.experimental import pallas as pl
from jax.experimental.pallas import tpu as pltpu
from jax.experimental.pallas import tpu_sc as plsc

VOCAB = 1000000
SEQ_LEN = 200
EMBED_DIM = 32
BATCH = 4096

NUM_CORES = 2
NUM_SUBCORES = 16
NUM_WORKERS = NUM_CORES * NUM_SUBCORES

LANES = 16
BB = 128
NBLK = BATCH // BB
DB = 8
NG = EMBED_DIM // DB
GRP = 4
NGRP = NBLK // GRP
NPAIR = NGRP // 2

L_BASE = SEQ_LEN // NUM_WORKERS
L_EXTRA = SEQ_LEN % NUM_WORKERS

OUT_ROWS = SEQ_LEN * NG * NBLK


def _body(idx_hbm, tok_hbm, pos_hbm, out_hbm,
          idx_v, r0_v, r1_v, o0_v, o1_v, pos_v, psplat_v,
          gsem0, gsem1, osem0, osem1):
    wid = lax.axis_index("s") * NUM_CORES + lax.axis_index("c")
    pltpu.sync_copy(pos_hbm, pos_v)

    lo = wid * L_BASE + jnp.minimum(wid, L_EXTRA)
    cnt = L_BASE + jnp.where(wid < L_EXTRA, 1, 0)

    iota = lax.iota(jnp.int32, LANES)
    zeros16 = jnp.zeros((LANES,), jnp.int32)
    row_vecs = [iota + (hh * LANES) for hh in range(BB // LANES)]
    d_vecs = [jnp.full((LANES,), d, jnp.int32) for d in range(EMBED_DIM)]

    r_bufs = (r0_v, r1_v)
    o_bufs = (o0_v, o1_v)
    gsems = (gsem0, gsem1)
    osems = (osem0, osem1)

    def fire_group(g, b):
        for j in range(GRP):
            pltpu.async_copy(
                tok_hbm.at[idx_v.at[g * GRP + j]], r_bufs[b].at[j], gsems[b]
            )

    def drain_gathers(b):
        for j in range(GRP):
            pltpu.make_async_copy(
                tok_hbm.at[pl.ds(0, BB)], r_bufs[b].at[j], gsems[b]
            ).wait()

    def drain_outs(b):
        for g_ in range(NG):
            pltpu.make_async_copy(
                out_hbm.at[pl.ds(0, GRP)], o_bufs[b].at[g_], osems[b]
            ).wait()

    def l_body(l, _):
        pltpu.sync_copy(idx_hbm.at[l], idx_v)
        l_vec = zeros16 + l

        for d in range(EMBED_DIM):
            psplat_v[d, :] = plsc.load_gather(pos_v, [l_vec, d_vecs[d]])

        fire_group(0, 0)
        fire_group(1, 1)

        def pair_body(p, carry):
            for b in range(2):
                g = 2 * p + b
                rbuf, obuf = r_bufs[b], o_bufs[b]
                drain_gathers(b)

                @pl.when(p >= 1)
                def _():
                    drain_outs(b)

                @plsc.parallel_loop(0, GRP * EMBED_DIM, unroll=2)
                def _(t):
                    cc = lax.shift_right_logical(t, 5)
                    d = lax.bitwise_and(t, EMBED_DIM - 1)
                    g_ = lax.shift_right_logical(d, 3)
                    dd = lax.bitwise_and(d, DB - 1)
                    rblk = rbuf.at[cc]
                    pv = psplat_v[d, :]
                    d_vec = zeros16 + d
                    for hh in range(BB // LANES):
                        vals = plsc.load_gather(rblk, [row_vecs[hh], d_vec])
                        obuf[g_, cc, dd, pl.ds(hh * LANES, LANES)] = vals + pv

                for g_ in range(NG):
                    row0 = (l * NG + g_) * NBLK + g * GRP
                    pltpu.async_copy(
                        obuf.at[g_], out_hbm.at[pl.ds(row0, GRP)], osems[b]
                    )

                @pl.when(p < NPAIR - 1)
                def _():
                    fire_group(g + 2, b)

            return carry

        lax.fori_loop(0, NPAIR, pair_body, 0)
        drain_outs(0)
        drain_outs(1)
        return _

    lax.fori_loop(lo, lo + cnt, l_body, 0)


_mesh = plsc.VectorSubcoreMesh(core_axis_name="c", subcore_axis_name="s")

_sc_call = functools.partial(
    pl.kernel,
    out_type=jax.ShapeDtypeStruct((OUT_ROWS, DB, BB), jnp.float32),
    mesh=_mesh,
    scratch_types=[
        pltpu.VMEM((NBLK, BB), jnp.int32),
        pltpu.VMEM((GRP, BB, EMBED_DIM), jnp.float32),
        pltpu.VMEM((GRP, BB, EMBED_DIM), jnp.float32),
        pltpu.VMEM((NG, GRP, DB, BB), jnp.float32),
        pltpu.VMEM((NG, GRP, DB, BB), jnp.float32),
        pltpu.VMEM((SEQ_LEN, EMBED_DIM), jnp.float32),
        pltpu.VMEM((EMBED_DIM, LANES), jnp.float32),
        pltpu.SemaphoreType.DMA,
        pltpu.SemaphoreType.DMA,
        pltpu.SemaphoreType.DMA,
        pltpu.SemaphoreType.DMA,
    ],
    compiler_params=pltpu.CompilerParams(
        use_tc_tiling_on_sc=False, needs_layout_passes=False
    ),
)


@jax.jit
def kernel(inputs, token_table, pos_table):
    idx = inputs.astype(jnp.int32).T.reshape(SEQ_LEN, NBLK, BB)
    tok_lin = lax.optimization_barrier(
        token_table.reshape(VOCAB // 4, 4 * EMBED_DIM)
    ).reshape(VOCAB, EMBED_DIM)
    o3 = _sc_call(_body)(idx, tok_lin, pos_table)
    o5 = o3.reshape(SEQ_LEN, NG, NBLK, DB, BB)
    out = o5.transpose(0, 1, 3, 2, 4).reshape(SEQ_LEN, EMBED_DIM, BATCH)
    return out.transpose(2, 0, 1)


# speedup vs baseline: 1.5953x; 1.0220x over previous
import functools

import jax
import jax.numpy as jnp
from jax import lax
from jax.experimental import pallas as pl
from jax.experimental.pallas import tpu as pltpu
from jax.experimental.pallas import tpu_sc as plsc

VOCAB = 1000000
SEQ_LEN = 200
EMBED_DIM = 32
BATCH = 4096

NUM_CORES = 2
NUM_SUBCORES = 16
NUM_WORKERS = NUM_CORES * NUM_SUBCORES

LANES = 16
BB = 128
HB = BB // 2
NBLK = BATCH // BB
HH_N = BB // LANES
DB = 8
NG = EMBED_DIM // DB
GRP = 4
NGRP = NBLK // GRP

L_BASE = SEQ_LEN // NUM_WORKERS
L_EXTRA = SEQ_LEN % NUM_WORKERS
L_MAX = L_BASE + 1
SEQ_PAD = 208

OUT_ROWS = SEQ_LEN * NG * NBLK


def _body(idx_hbm, tok_hbm, pos_hbm, out_hbm,
          idx_v, r0_v, r1_v, o0_v, o1_v, pos_v, psplat_v,
          gsem0, gsem1, osem0, osem1):
    wid = lax.axis_index("s") * NUM_CORES + lax.axis_index("c")

    lo = wid * L_BASE + jnp.minimum(wid, L_EXTRA)
    cnt = L_BASE + jnp.where(wid < L_EXTRA, 1, 0)
    n_groups = cnt * NGRP

    pltpu.sync_copy(idx_hbm.at[pl.ds(lo, L_MAX)], idx_v)
    pltpu.sync_copy(pos_hbm.at[pl.ds(lo, DB)], pos_v)

    iota = lax.iota(jnp.int32, LANES)
    zeros16 = jnp.zeros((LANES,), jnp.int32)
    d_vecs = [jnp.full((LANES,), d, jnp.int32) for d in range(EMBED_DIM)]

    r_bufs = (r0_v, r1_v)
    o_bufs = (o0_v, o1_v)
    gsems = (gsem0, gsem1)
    osems = (osem0, osem1)

    def fire_group(gg, b):
        l_off = lax.shift_right_logical(gg, 3)
        g = lax.bitwise_and(gg, NGRP - 1)
        for j in range(GRP):
            idx_row = idx_v.at[l_off, g * GRP + j]
            for h in range(2):
                pltpu.async_copy(
                    tok_hbm.at[idx_row.at[pl.ds(h * HB, HB)]],
                    r_bufs[b].at[j, pl.ds(h * HB, HB)],
                    gsems[b],
                )

    def drain_gathers(b):
        for j in range(GRP):
            for h in range(2):
                pltpu.make_async_copy(
                    tok_hbm.at[pl.ds(0, HB)],
                    r_bufs[b].at[j, pl.ds(h * HB, HB)],
                    gsems[b],
                ).wait()

    def drain_outs(b):
        for g_ in range(NG):
            pltpu.make_async_copy(
                out_hbm.at[pl.ds(0, GRP)], o_bufs[b].at[g_], osems[b]
            ).wait()

    fire_group(0, 0)
    fire_group(1, 1)

    def pair_body(t, carry):
        for b in range(2):
            gg = 2 * t + b
            l_off = lax.shift_right_logical(gg, 3)
            g = lax.bitwise_and(gg, NGRP - 1)
            l = lo + l_off
            rbuf, obuf = r_bufs[b], o_bufs[b]
            drain_gathers(b)

            @pl.when(t >= 1)
            def _():
                drain_outs(b)

            if b == 0:
                @pl.when(g == 0)
                def _():
                    l_vec = zeros16 + l_off
                    for d in range(EMBED_DIM):
                        psplat_v[d, :] = plsc.load_gather(
                            pos_v, [l_vec, d_vecs[d]]
                        )

            @plsc.parallel_loop(0, GRP * EMBED_DIM, unroll=2)
            def _(u):
                cc = lax.shift_right_logical(u, 5)
                d = lax.bitwise_and(u, EMBED_DIM - 1)
                g_ = lax.shift_right_logical(d, 3)
                dd = lax.bitwise_and(d, DB - 1)
                rblk = rbuf.at[cc]
                pv = psplat_v[d, :]
                d_vec = zeros16 + d
                for hh in range(HH_N):
                    vals = plsc.load_gather(rblk, [iota + hh * LANES, d_vec])
                    obuf[g_, cc, dd, pl.ds(hh * LANES, LANES)] = vals + pv

            for g_ in range(NG):
                row0 = (l * NG + g_) * NBLK + g * GRP
                pltpu.async_copy(
                    obuf.at[g_], out_hbm.at[pl.ds(row0, GRP)], osems[b]
                )

            @pl.when(gg + 2 < n_groups)
            def _():
                fire_group(gg + 2, b)

        return carry

    lax.fori_loop(0, cnt * (NGRP // 2), pair_body, 0)
    drain_outs(0)
    drain_outs(1)


_mesh = plsc.VectorSubcoreMesh(core_axis_name="c", subcore_axis_name="s")

_sc_call = functools.partial(
    pl.kernel,
    out_type=jax.ShapeDtypeStruct((OUT_ROWS, DB, BB), jnp.float32),
    mesh=_mesh,
    scratch_types=[
        pltpu.VMEM((L_MAX, NBLK, BB), jnp.int32),
        pltpu.VMEM((GRP, BB, EMBED_DIM), jnp.float32),
        pltpu.VMEM((GRP, BB, EMBED_DIM), jnp.float32),
        pltpu.VMEM((NG, GRP, DB, BB), jnp.float32),
        pltpu.VMEM((NG, GRP, DB, BB), jnp.float32),
        pltpu.VMEM((DB, EMBED_DIM), jnp.float32),
        pltpu.VMEM((EMBED_DIM, LANES), jnp.float32),
        pltpu.SemaphoreType.DMA,
        pltpu.SemaphoreType.DMA,
        pltpu.SemaphoreType.DMA,
        pltpu.SemaphoreType.DMA,
    ],
    compiler_params=pltpu.CompilerParams(
        use_tc_tiling_on_sc=False, needs_layout_passes=False
    ),
)


@jax.jit
def kernel(inputs, token_table, pos_table):
    idx = inputs.astype(jnp.int32).T.reshape(SEQ_LEN, NBLK, BB)
    idx = jnp.pad(idx, ((0, SEQ_PAD - SEQ_LEN), (0, 0), (0, 0)))
    pos = jnp.pad(pos_table, ((0, SEQ_PAD - SEQ_LEN), (0, 0)))
    o3 = _sc_call(_body)(idx, token_table, pos)
    o5 = o3.reshape(SEQ_LEN, NG, NBLK, DB, BB)
    out = o5.transpose(0, 1, 3, 2, 4).reshape(SEQ_LEN, EMBED_DIM, BATCH)
    return out.transpose(2, 0, 1)


# speedup vs baseline: 2.1188x; 1.3282x over previous
import functools

import jax
import jax.numpy as jnp
from jax import lax
from jax.experimental import pallas as pl
from jax.experimental.pallas import tpu as pltpu
from jax.experimental.pallas import tpu_sc as plsc

VOCAB = 1000000
SEQ_LEN = 200
EMBED_DIM = 32
BATCH = 4096

NUM_CORES = 2
NUM_SUBCORES = 16
NUM_WORKERS = NUM_CORES * NUM_SUBCORES

LANES = 16
BB = 128
HB = BB // 2
NBLK = BATCH // BB
HH_N = BB // LANES
DB = 8
NG = EMBED_DIM // DB
GRP = 4
NGRP = NBLK // GRP

L_BASE = SEQ_LEN // NUM_WORKERS
L_EXTRA = SEQ_LEN % NUM_WORKERS
L_MAX = L_BASE + 1
SEQ_PAD = 208

OUT_ROWS = SEQ_LEN * NG * NBLK


def _body(idx_hbm, tok_hbm, pos_hbm, out_hbm,
          idx_v, r0_v, r1_v, o0_v, o1_v, pos_v, psplat_v,
          gsem0, gsem1, osem0, osem1):
    wid = lax.axis_index("s") * NUM_CORES + lax.axis_index("c")

    lo = wid * L_BASE + jnp.minimum(wid, L_EXTRA)
    cnt = L_BASE + jnp.where(wid < L_EXTRA, 1, 0)
    n_groups = cnt * NGRP

    pltpu.sync_copy(idx_hbm.at[pl.ds(lo, L_MAX)], idx_v)
    pltpu.sync_copy(pos_hbm.at[pl.ds(lo, DB)], pos_v)

    iota = lax.iota(jnp.int32, LANES)
    zeros16 = jnp.zeros((LANES,), jnp.int32)
    col_c = [[16 * t + ((iota + k) & 15) for k in range(LANES)]
             for t in range(2)]
    g_c = [[lax.shift_right_logical(c, 3) for c in cols] for cols in col_c]
    dd_c = [[lax.bitwise_and(c, DB - 1) for c in cols] for cols in col_c]

    r_bufs = (r0_v, r1_v)
    o_bufs = (o0_v, o1_v)
    gsems = (gsem0, gsem1)
    osems = (osem0, osem1)

    def fire_group(gg, b):
        l_off = lax.shift_right_logical(gg, 3)
        g = lax.bitwise_and(gg, NGRP - 1)
        for j in range(GRP):
            idx_row = idx_v.at[l_off, g * GRP + j]
            for h in range(2):
                pltpu.async_copy(
                    tok_hbm.at[idx_row.at[pl.ds(h * HB, HB)]],
                    r_bufs[b].at[j, pl.ds(h * HB, HB)],
                    gsems[b],
                )

    def drain_gathers(b):
        for j in range(GRP):
            for h in range(2):
                pltpu.make_async_copy(
                    tok_hbm.at[pl.ds(0, HB)],
                    r_bufs[b].at[j, pl.ds(h * HB, HB)],
                    gsems[b],
                ).wait()

    def drain_outs(b):
        for g_ in range(NG):
            pltpu.make_async_copy(
                out_hbm.at[pl.ds(0, GRP)], o_bufs[b].at[g_], osems[b]
            ).wait()

    fire_group(0, 0)
    fire_group(1, 1)

    def pair_body(t, carry):
        for b in range(2):
            gg = 2 * t + b
            l_off = lax.shift_right_logical(gg, 3)
            g = lax.bitwise_and(gg, NGRP - 1)
            l = lo + l_off
            rbuf, obuf = r_bufs[b], o_bufs[b]
            drain_gathers(b)

            @pl.when(t >= 1)
            def _():
                drain_outs(b)

            if b == 0:
                @pl.when(g == 0)
                def _():
                    l_vec = zeros16 + l_off
                    for t in range(2):
                        for k in range(LANES):
                            psplat_v[16 * t + k, :] = plsc.load_gather(
                                pos_v, [l_vec, col_c[t][k]]
                            )

            @plsc.parallel_loop(0, GRP * HH_N, unroll=1)
            def _(u):
                cc = lax.shift_right_logical(u, 3)
                hh = lax.bitwise_and(u, HH_N - 1)
                rblk = rbuf.at[cc]
                row_vec = iota + hh * LANES
                cc_vec = zeros16 + cc
                for t in range(2):
                    for k in range(LANES):
                        vals = plsc.load_gather(rblk, [row_vec, col_c[t][k]])
                        vals = vals + psplat_v[16 * t + k, :]
                        plsc.store_scatter(
                            obuf,
                            [g_c[t][k], cc_vec, dd_c[t][k], row_vec],
                            vals,
                        )

            for g_ in range(NG):
                row0 = (l * NG + g_) * NBLK + g * GRP
                pltpu.async_copy(
                    obuf.at[g_], out_hbm.at[pl.ds(row0, GRP)], osems[b]
                )

            @pl.when(gg + 2 < n_groups)
            def _():
                fire_group(gg + 2, b)

        return carry

    lax.fori_loop(0, cnt * (NGRP // 2), pair_body, 0)
    drain_outs(0)
    drain_outs(1)


_mesh = plsc.VectorSubcoreMesh(core_axis_name="c", subcore_axis_name="s")

_sc_call = functools.partial(
    pl.kernel,
    out_type=jax.ShapeDtypeStruct((OUT_ROWS, DB, BB), jnp.float32),
    mesh=_mesh,
    scratch_types=[
        pltpu.VMEM((L_MAX, NBLK, BB), jnp.int32),
        pltpu.VMEM((GRP, BB, EMBED_DIM), jnp.float32),
        pltpu.VMEM((GRP, BB, EMBED_DIM), jnp.float32),
        pltpu.VMEM((NG, GRP, DB, BB), jnp.float32),
        pltpu.VMEM((NG, GRP, DB, BB), jnp.float32),
        pltpu.VMEM((DB, EMBED_DIM), jnp.float32),
        pltpu.VMEM((EMBED_DIM, LANES), jnp.float32),
        pltpu.SemaphoreType.DMA,
        pltpu.SemaphoreType.DMA,
        pltpu.SemaphoreType.DMA,
        pltpu.SemaphoreType.DMA,
    ],
    compiler_params=pltpu.CompilerParams(
        use_tc_tiling_on_sc=False, needs_layout_passes=False
    ),
)


@jax.jit
def kernel(inputs, token_table, pos_table):
    idx = inputs.astype(jnp.int32).T.reshape(SEQ_LEN, NBLK, BB)
    idx = jnp.pad(idx, ((0, SEQ_PAD - SEQ_LEN), (0, 0), (0, 0)))
    pos = jnp.pad(pos_table, ((0, SEQ_PAD - SEQ_LEN), (0, 0)))
    o3 = _sc_call(_body)(idx, token_table, pos)
    o5 = o3.reshape(SEQ_LEN, NG, NBLK, DB, BB)
    out = o5.transpose(0, 1, 3, 2, 4).reshape(SEQ_LEN, EMBED_DIM, BATCH)
    return out.transpose(2, 0, 1)
